# Initial kernel scaffold; baseline (speedup 1.0000x reference)
import jax
import jax.numpy as jnp
from jax.experimental import pallas as pl


def kernel(x, edge_index, batch, global_features, W_emb, b_emb, W1, b1, W2, b2, Wg, bg, Wh1, bh1, Wh2, bh2):
    raise NotImplementedError("write your pallas kernel here")


# speedup vs baseline: 13.6020x; 13.6020x over previous
import functools

import jax
import jax.numpy as jnp
from jax import lax
from jax.experimental import pallas as pl
from jax.experimental.pallas import tpu as pltpu
from jax.experimental.pallas import tpu_sc as plsc

N = 10000
E = 320000
HID = 128
NACT = 64
NB = 64
NCORES = 2
NSUB = 16
NW = NCORES * NSUB
EPW = E // NW
CHUNK = 80
NCHUNK = EPW // CHUNK
RPT = 632
NPAD = RPT * NSUB

_mesh = plsc.VectorSubcoreMesh(core_axis_name="c", subcore_axis_name="s")


@functools.partial(
    pl.kernel,
    out_type=jax.ShapeDtypeStruct((NW, N), jnp.float32),
    mesh=_mesh,
    scratch_types=[
        pltpu.VMEM((EPW,), jnp.int32),
        pltpu.VMEM((N,), jnp.float32),
    ],
    compiler_params=pltpu.CompilerParams(needs_layout_passes=False),
)
def _sc_degree(dst_hbm, out_hbm, idx_v, hist_v):
    cid = lax.axis_index("c")
    sid = lax.axis_index("s")
    wid = sid * NCORES + cid

    pltpu.sync_copy(dst_hbm.at[pl.ds(wid * EPW, EPW)], idx_v)

    def zero_body(i, carry):
        hist_v[pl.ds(i * 16, 16)] = jnp.zeros((16,), jnp.float32)
        return carry

    lax.fori_loop(0, N // 16, zero_body, 0)

    ones = jnp.ones((16,), jnp.float32)

    def body(i, carry):
        idx = idx_v[pl.ds(i * 16, 16)]
        plsc.addupdate_scatter(hist_v, [idx], ones)
        return carry

    lax.fori_loop(0, EPW // 16, body, 0)
    pltpu.sync_copy(hist_v, out_hbm.at[wid])


@functools.partial(
    pl.kernel,
    out_type=jax.ShapeDtypeStruct((NCORES, NPAD, HID), jnp.float32),
    mesh=_mesh,
    scratch_types=[
        pltpu.VMEM((CHUNK,), jnp.int32),
        pltpu.VMEM((CHUNK,), jnp.int32),
        pltpu.VMEM((CHUNK, HID), jnp.float32),
        pltpu.VMEM_SHARED((NPAD, HID), jnp.float32),
        pltpu.SemaphoreType.DMA,
    ],
)
def _sc_edge_agg(y_hbm, src_hbm, dst_hbm, zeros_hbm, out_hbm,
                 src_v, dst_v, rows_v, accum_sh, sem):
    cid = lax.axis_index("c")
    sid = lax.axis_index("s")
    wid = sid * NCORES + cid
    base = wid * EPW

    r0 = sid * RPT
    pltpu.sync_copy(zeros_hbm.at[pl.ds(r0, RPT)], accum_sh.at[pl.ds(r0, RPT)])
    plsc.subcore_barrier()

    def body(i, carry):
        e0 = base + i * CHUNK
        pltpu.sync_copy(src_hbm.at[pl.ds(e0, CHUNK)], src_v)
        pltpu.sync_copy(dst_hbm.at[pl.ds(e0, CHUNK)], dst_v)
        pltpu.async_copy(y_hbm.at[src_v], rows_v, sem).wait()
        pltpu.sync_copy(rows_v, accum_sh.at[dst_v], add=True)
        return carry

    lax.fori_loop(0, NCHUNK, body, 0)
    plsc.subcore_barrier()
    pltpu.sync_copy(accum_sh.at[pl.ds(r0, RPT)],
                    out_hbm.at[cid].at[pl.ds(r0, RPT)])


def _tc_pre_body(degp_ref, x_ref, wemb_ref, bemb_ref, w1_ref, dinv_ref, y_ref):
    deg = jnp.sum(degp_ref[...], axis=0) + 1.0
    dinv = lax.rsqrt(deg)
    dinv_ref[...] = dinv[:, None]
    h0 = jnp.maximum(
        jnp.dot(x_ref[...], wemb_ref[...],
                preferred_element_type=jnp.float32) + bemb_ref[...][None, :],
        0.0)
    xw = jnp.dot(h0, w1_ref[...], preferred_element_type=jnp.float32)
    y_ref[...] = xw * dinv[:, None]


def _tc_mid_body(acc_ref, yprev_ref, dinv_ref, bprev_ref, wnext_ref, y_ref):
    dinv = dinv_ref[...]
    acc = (acc_ref[0] + acc_ref[1])[:N]
    h = jnp.maximum(
        dinv * (acc + yprev_ref[...]) + bprev_ref[...][None, :], 0.0)
    y_ref[...] = jnp.dot(h, wnext_ref[...],
                         preferred_element_type=jnp.float32) * dinv


def _tc_final_body(acc_ref, yprev_ref, dinv_ref, b2_ref, batch_ref, gf_ref,
                   wg_ref, bg_ref, wh1_ref, bh1_ref, wh2_ref, bh2_ref, q_ref):
    dinv = dinv_ref[...]
    acc = (acc_ref[0] + acc_ref[1])[:N]
    h = jnp.maximum(
        dinv * (acc + yprev_ref[...]) + b2_ref[...][None, :],
        0.0)
    gid = lax.broadcasted_iota(jnp.int32, (NB, N), 0)
    oh_t = (gid == batch_ref[...]).astype(jnp.float32)
    seg = jnp.dot(oh_t, h, preferred_element_type=jnp.float32)
    cnt = jnp.sum(oh_t, axis=1)
    ge = seg / jnp.maximum(cnt, 1.0)[:, None]
    gl = jnp.maximum(
        jnp.dot(gf_ref[...], wg_ref[...],
                preferred_element_type=jnp.float32) + bg_ref[...][None, :],
        0.0)
    comb = jnp.concatenate([ge, gl], axis=1)
    hh = jnp.maximum(
        jnp.dot(comb, wh1_ref[...],
                preferred_element_type=jnp.float32) + bh1_ref[...][None, :],
        0.0)
    q_ref[...] = (jnp.dot(hh, wh2_ref[...], preferred_element_type=jnp.float32)
                  + bh2_ref[...][None, :])


def _tc_call(body, out_shape, *args):
    return pl.pallas_call(body, out_shape=out_shape)(*args)


def kernel(x, edge_index, batch, global_features, W_emb, b_emb, W1, b1,
           W2, b2, Wg, bg, Wh1, bh1, Wh2, bh2):
    src = edge_index[0].astype(jnp.int32)
    dst = edge_index[1].astype(jnp.int32)
    zeros = jnp.zeros((NPAD, HID), jnp.float32)

    deg_parts = _sc_degree(dst)

    dinv, y1 = _tc_call(
        _tc_pre_body,
        (jax.ShapeDtypeStruct((N, 1), jnp.float32),
         jax.ShapeDtypeStruct((N, HID), jnp.float32)),
        deg_parts, x, W_emb, b_emb, W1)

    acc1 = _sc_edge_agg(y1, src, dst, zeros)

    y2 = _tc_call(
        _tc_mid_body,
        jax.ShapeDtypeStruct((N, HID), jnp.float32),
        acc1, y1, dinv, b1, W2)

    acc2 = _sc_edge_agg(y2, src, dst, zeros)

    q = _tc_call(
        _tc_final_body,
        jax.ShapeDtypeStruct((NB, NACT), jnp.float32),
        acc2, y2, dinv, b2, batch.astype(jnp.int32)[None, :],
        global_features, Wg, bg, Wh1, bh1, Wh2, bh2)
    return q


# speedup vs baseline: 28.5702x; 2.1004x over previous
import functools

import jax
import jax.numpy as jnp
from jax import lax
from jax.experimental import pallas as pl
from jax.experimental.pallas import tpu as pltpu
from jax.experimental.pallas import tpu_sc as plsc

N = 10000
E = 320000
HID = 128
NACT = 64
NB = 64
NCORES = 2
NSUB = 16
NW = NCORES * NSUB
EPW = E // NW
CHUNK = 80
NCHUNK = EPW // CHUNK
NSTAGE = 5
STAGE = NCHUNK // NSTAGE
RPT = 632
NPAD = RPT * NSUB

_mesh = plsc.VectorSubcoreMesh(core_axis_name="c", subcore_axis_name="s")


@functools.partial(
    pl.kernel,
    out_type=jax.ShapeDtypeStruct((NW, N), jnp.float32),
    mesh=_mesh,
    scratch_types=[
        pltpu.VMEM((EPW,), jnp.int32),
        pltpu.VMEM((N,), jnp.float32),
    ],
    compiler_params=pltpu.CompilerParams(needs_layout_passes=False),
)
def _sc_degree(dst_hbm, out_hbm, idx_v, hist_v):
    cid = lax.axis_index("c")
    sid = lax.axis_index("s")
    wid = sid * NCORES + cid

    pltpu.sync_copy(dst_hbm.at[pl.ds(wid * EPW, EPW)], idx_v)

    def zero_body(i, carry):
        hist_v[pl.ds(i * 16, 16)] = jnp.zeros((16,), jnp.float32)
        return carry

    lax.fori_loop(0, N // 16, zero_body, 0)

    ones = jnp.ones((16,), jnp.float32)

    def body(i, carry):
        idx = idx_v[pl.ds(i * 16, 16)]
        plsc.addupdate_scatter(hist_v, [idx], ones)
        return carry

    lax.fori_loop(0, EPW // 16, body, 0)
    pltpu.sync_copy(hist_v, out_hbm.at[wid])


@functools.partial(
    pl.kernel,
    out_type=jax.ShapeDtypeStruct((NCORES, NPAD, HID), jnp.float32),
    mesh=_mesh,
    scratch_types=[
        pltpu.VMEM((STAGE, CHUNK), jnp.int32),
        pltpu.VMEM((STAGE, CHUNK), jnp.int32),
        pltpu.VMEM((CHUNK, HID), jnp.float32),
        pltpu.VMEM((CHUNK, HID), jnp.float32),
        pltpu.VMEM_SHARED((NPAD, HID), jnp.float32),
        pltpu.SemaphoreType.DMA,
        pltpu.SemaphoreType.DMA,
    ],
)
def _sc_edge_agg(y_hbm, src_hbm, dst_hbm, zeros_hbm, out_hbm,
                 src_v, dst_v, rows0_v, rows1_v, accum_sh, sem0, sem1):
    cid = lax.axis_index("c")
    sid = lax.axis_index("s")
    wid = sid * NCORES + cid

    r0 = sid * RPT
    pltpu.sync_copy(zeros_hbm.at[pl.ds(r0, RPT)], accum_sh.at[pl.ds(r0, RPT)])
    plsc.subcore_barrier()

    def gather(i, buf, sem):
        return pltpu.async_copy(y_hbm.at[src_v.at[i]], buf, sem)

    def scatter(i, buf):
        pltpu.sync_copy(buf, accum_sh.at[dst_v.at[i]], add=True)

    def stage_body(s, carry):
        pltpu.sync_copy(src_hbm.at[wid, s], src_v)
        pltpu.sync_copy(dst_hbm.at[wid, s], dst_v)

        gather(0, rows0_v, sem0)

        def body(j, c):
            i2 = j * 2
            gather(i2 + 1, rows1_v, sem1)
            pltpu.make_async_copy(y_hbm.at[src_v.at[i2]], rows0_v, sem0).wait()
            scatter(i2, rows0_v)
            gather(i2 + 2, rows0_v, sem0)
            pltpu.make_async_copy(y_hbm.at[src_v.at[i2 + 1]], rows1_v,
                                  sem1).wait()
            scatter(i2 + 1, rows1_v)
            return c

        lax.fori_loop(0, (STAGE - 1) // 2, body, 0)
        pltpu.make_async_copy(y_hbm.at[src_v.at[STAGE - 1]], rows0_v,
                              sem0).wait()
        scatter(STAGE - 1, rows0_v)
        return carry

    lax.fori_loop(0, NSTAGE, stage_body, 0)

    plsc.subcore_barrier()
    pltpu.sync_copy(accum_sh.at[pl.ds(r0, RPT)],
                    out_hbm.at[cid].at[pl.ds(r0, RPT)])


def _tc_pre_body(degp_ref, x_ref, wemb_ref, bemb_ref, w1_ref, dinv_ref, y_ref):
    deg = jnp.sum(degp_ref[...], axis=0) + 1.0
    dinv = lax.rsqrt(deg)
    dinv_ref[...] = dinv[:, None]
    h0 = jnp.maximum(
        jnp.dot(x_ref[...], wemb_ref[...],
                preferred_element_type=jnp.float32) + bemb_ref[...][None, :],
        0.0)
    xw = jnp.dot(h0, w1_ref[...], preferred_element_type=jnp.float32)
    y_ref[...] = xw * dinv[:, None]


def _tc_mid_body(acc_ref, yprev_ref, dinv_ref, bprev_ref, wnext_ref, y_ref):
    dinv = dinv_ref[...]
    acc = (acc_ref[0] + acc_ref[1])[:N]
    h = jnp.maximum(
        dinv * (acc + yprev_ref[...]) + bprev_ref[...][None, :], 0.0)
    y_ref[...] = jnp.dot(h, wnext_ref[...],
                         preferred_element_type=jnp.float32) * dinv


def _tc_final_body(acc_ref, yprev_ref, dinv_ref, b2_ref, batch_ref, gf_ref,
                   wg_ref, bg_ref, wh1_ref, bh1_ref, wh2_ref, bh2_ref, q_ref):
    dinv = dinv_ref[...]
    acc = (acc_ref[0] + acc_ref[1])[:N]
    h = jnp.maximum(
        dinv * (acc + yprev_ref[...]) + b2_ref[...][None, :],
        0.0)
    gid = lax.broadcasted_iota(jnp.int32, (NB, N), 0)
    oh_t = (gid == batch_ref[...]).astype(jnp.float32)
    seg = jnp.dot(oh_t, h, preferred_element_type=jnp.float32)
    cnt = jnp.sum(oh_t, axis=1)
    ge = seg / jnp.maximum(cnt, 1.0)[:, None]
    gl = jnp.maximum(
        jnp.dot(gf_ref[...], wg_ref[...],
                preferred_element_type=jnp.float32) + bg_ref[...][None, :],
        0.0)
    comb = jnp.concatenate([ge, gl], axis=1)
    hh = jnp.maximum(
        jnp.dot(comb, wh1_ref[...],
                preferred_element_type=jnp.float32) + bh1_ref[...][None, :],
        0.0)
    q_ref[...] = (jnp.dot(hh, wh2_ref[...], preferred_element_type=jnp.float32)
                  + bh2_ref[...][None, :])


def _tc_call(body, out_shape, *args):
    return pl.pallas_call(body, out_shape=out_shape)(*args)


def kernel(x, edge_index, batch, global_features, W_emb, b_emb, W1, b1,
           W2, b2, Wg, bg, Wh1, bh1, Wh2, bh2):
    src = edge_index[0].astype(jnp.int32)
    dst = edge_index[1].astype(jnp.int32)
    src3 = src.reshape(NW, NSTAGE, STAGE, CHUNK)
    dst3 = dst.reshape(NW, NSTAGE, STAGE, CHUNK)
    zeros = jnp.zeros((NPAD, HID), jnp.float32)

    deg_parts = _sc_degree(dst)

    dinv, y1 = _tc_call(
        _tc_pre_body,
        (jax.ShapeDtypeStruct((N, 1), jnp.float32),
         jax.ShapeDtypeStruct((N, HID), jnp.float32)),
        deg_parts, x, W_emb, b_emb, W1)

    acc1 = _sc_edge_agg(y1, src3, dst3, zeros)

    y2 = _tc_call(
        _tc_mid_body,
        jax.ShapeDtypeStruct((N, HID), jnp.float32),
        acc1, y1, dinv, b1, W2)

    acc2 = _sc_edge_agg(y2, src3, dst3, zeros)

    q = _tc_call(
        _tc_final_body,
        jax.ShapeDtypeStruct((NB, NACT), jnp.float32),
        acc2, y2, dinv, b2, batch.astype(jnp.int32)[None, :],
        global_features, Wg, bg, Wh1, bh1, Wh2, bh2)
    return q


# speedup vs baseline: 31.9906x; 1.1197x over previous
import functools

import jax
import jax.numpy as jnp
from jax import lax
from jax.experimental import pallas as pl
from jax.experimental.pallas import tpu as pltpu
from jax.experimental.pallas import tpu_sc as plsc

N = 10000
E = 320000
HID = 128
NACT = 64
NB = 64
NCORES = 2
NSUB = 16
NW = NCORES * NSUB
EPW = E // NW
CHUNK = 80
NCHUNK = EPW // CHUNK
NSTAGE = 5
STAGE = NCHUNK // NSTAGE
RPT = 632
NPAD = RPT * NSUB

_mesh = plsc.VectorSubcoreMesh(core_axis_name="c", subcore_axis_name="s")


@functools.partial(
    pl.kernel,
    out_type=jax.ShapeDtypeStruct((NW, N), jnp.float32),
    mesh=_mesh,
    scratch_types=[
        pltpu.VMEM((EPW,), jnp.int32),
        pltpu.VMEM((N,), jnp.float32),
    ],
    compiler_params=pltpu.CompilerParams(needs_layout_passes=False),
)
def _sc_degree(dst_hbm, out_hbm, idx_v, hist_v):
    cid = lax.axis_index("c")
    sid = lax.axis_index("s")
    wid = sid * NCORES + cid

    pltpu.sync_copy(dst_hbm.at[pl.ds(wid * EPW, EPW)], idx_v)

    def zero_body(i, carry):
        hist_v[pl.ds(i * 16, 16)] = jnp.zeros((16,), jnp.float32)
        return carry

    lax.fori_loop(0, N // 16, zero_body, 0)

    ones = jnp.ones((16,), jnp.float32)

    def body(i, carry):
        idx = idx_v[pl.ds(i * 16, 16)]
        plsc.addupdate_scatter(hist_v, [idx], ones)
        return carry

    lax.fori_loop(0, EPW // 16, body, 0)
    pltpu.sync_copy(hist_v, out_hbm.at[wid])


@functools.partial(
    pl.kernel,
    out_type=jax.ShapeDtypeStruct((NCORES, NPAD, HID), jnp.float32),
    mesh=_mesh,
    scratch_types=[
        pltpu.VMEM((STAGE, CHUNK), jnp.int32),
        pltpu.VMEM((STAGE, CHUNK), jnp.int32),
        [pltpu.VMEM((CHUNK, HID), jnp.float32) for _ in range(3)],
        pltpu.VMEM_SHARED((NPAD, HID), jnp.float32),
        [pltpu.SemaphoreType.DMA for _ in range(3)],
        [pltpu.SemaphoreType.DMA for _ in range(3)],
    ],
)
def _sc_edge_agg(y_hbm, src_hbm, dst_hbm, zeros_hbm, out_hbm,
                 src_v, dst_v, rows, accum_sh, gsem, ssem):
    cid = lax.axis_index("c")
    sid = lax.axis_index("s")
    wid = sid * NCORES + cid

    r0 = sid * RPT
    pltpu.sync_copy(zeros_hbm.at[pl.ds(r0, RPT)], accum_sh.at[pl.ds(r0, RPT)])
    plsc.subcore_barrier()

    def gather(i, b):
        pltpu.async_copy(y_hbm.at[src_v.at[i]], rows[b], gsem[b])

    def gather_wait(i, b):
        pltpu.make_async_copy(y_hbm.at[src_v.at[i]], rows[b], gsem[b]).wait()

    def scatter(i, b):
        pltpu.async_copy(rows[b], accum_sh.at[dst_v.at[i]], ssem[b], add=True)

    def scatter_wait(i, b):
        pltpu.make_async_copy(rows[b], accum_sh.at[dst_v.at[i]],
                              ssem[b]).wait()

    def stage_body(s, carry):
        pltpu.sync_copy(src_hbm.at[wid, s], src_v)
        pltpu.sync_copy(dst_hbm.at[wid, s], dst_v)

        gather(0, 0)
        gather(1, 1)
        gather_wait(0, 0)
        scatter(0, 0)
        gather(2, 2)

        def step(i, b):
            gather_wait(i, b)
            scatter(i, b)
            bp = (b + 2) % 3
            scatter_wait(i - 1, bp)
            gather(i + 2, bp)

        def body(j, c):
            for t in range(3):
                i = j * 3 + 1 + t
                step(i, (1 + t) % 3)
            return c

        lax.fori_loop(0, (STAGE - 4) // 3, body, 0)
        step(STAGE - 3, (STAGE - 3) % 3)
        for i in (STAGE - 2, STAGE - 1):
            b = i % 3
            gather_wait(i, b)
            scatter(i, b)
            scatter_wait(i - 1, (i - 1) % 3)
        scatter_wait(STAGE - 1, (STAGE - 1) % 3)
        return carry

    lax.fori_loop(0, NSTAGE, stage_body, 0)

    plsc.subcore_barrier()
    pltpu.sync_copy(accum_sh.at[pl.ds(r0, RPT)],
                    out_hbm.at[cid].at[pl.ds(r0, RPT)])


def _tc_pre_body(degp_ref, x_ref, wemb_ref, bemb_ref, w1_ref, dinv_ref, y_ref):
    deg = jnp.sum(degp_ref[...], axis=0) + 1.0
    dinv = lax.rsqrt(deg)
    dinv_ref[...] = dinv[:, None]
    h0 = jnp.maximum(
        jnp.dot(x_ref[...], wemb_ref[...],
                preferred_element_type=jnp.float32) + bemb_ref[...][None, :],
        0.0)
    xw = jnp.dot(h0, w1_ref[...], preferred_element_type=jnp.float32)
    y_ref[...] = xw * dinv[:, None]


def _tc_mid_body(acc_ref, yprev_ref, dinv_ref, bprev_ref, wnext_ref, y_ref):
    dinv = dinv_ref[...]
    acc = (acc_ref[0] + acc_ref[1])[:N]
    h = jnp.maximum(
        dinv * (acc + yprev_ref[...]) + bprev_ref[...][None, :], 0.0)
    y_ref[...] = jnp.dot(h, wnext_ref[...],
                         preferred_element_type=jnp.float32) * dinv


def _tc_final_body(acc_ref, yprev_ref, dinv_ref, b2_ref, batch_ref, gf_ref,
                   wg_ref, bg_ref, wh1_ref, bh1_ref, wh2_ref, bh2_ref, q_ref):
    dinv = dinv_ref[...]
    acc = (acc_ref[0] + acc_ref[1])[:N]
    h = jnp.maximum(
        dinv * (acc + yprev_ref[...]) + b2_ref[...][None, :],
        0.0)
    gid = lax.broadcasted_iota(jnp.int32, (NB, N), 0)
    oh_t = (gid == batch_ref[...]).astype(jnp.float32)
    seg = jnp.dot(oh_t, h, preferred_element_type=jnp.float32)
    cnt = jnp.sum(oh_t, axis=1)
    ge = seg / jnp.maximum(cnt, 1.0)[:, None]
    gl = jnp.maximum(
        jnp.dot(gf_ref[...], wg_ref[...],
                preferred_element_type=jnp.float32) + bg_ref[...][None, :],
        0.0)
    comb = jnp.concatenate([ge, gl], axis=1)
    hh = jnp.maximum(
        jnp.dot(comb, wh1_ref[...],
                preferred_element_type=jnp.float32) + bh1_ref[...][None, :],
        0.0)
    q_ref[...] = (jnp.dot(hh, wh2_ref[...], preferred_element_type=jnp.float32)
                  + bh2_ref[...][None, :])


def _tc_call(body, out_shape, *args):
    return pl.pallas_call(body, out_shape=out_shape)(*args)


def kernel(x, edge_index, batch, global_features, W_emb, b_emb, W1, b1,
           W2, b2, Wg, bg, Wh1, bh1, Wh2, bh2):
    src = edge_index[0].astype(jnp.int32)
    dst = edge_index[1].astype(jnp.int32)
    src3 = src.reshape(NW, NSTAGE, STAGE, CHUNK)
    dst3 = dst.reshape(NW, NSTAGE, STAGE, CHUNK)
    zeros = jnp.zeros((NPAD, HID), jnp.float32)

    deg_parts = _sc_degree(dst)

    dinv, y1 = _tc_call(
        _tc_pre_body,
        (jax.ShapeDtypeStruct((N, 1), jnp.float32),
         jax.ShapeDtypeStruct((N, HID), jnp.float32)),
        deg_parts, x, W_emb, b_emb, W1)

    acc1 = _sc_edge_agg(y1, src3, dst3, zeros)

    y2 = _tc_call(
        _tc_mid_body,
        jax.ShapeDtypeStruct((N, HID), jnp.float32),
        acc1, y1, dinv, b1, W2)

    acc2 = _sc_edge_agg(y2, src3, dst3, zeros)

    q = _tc_call(
        _tc_final_body,
        jax.ShapeDtypeStruct((NB, NACT), jnp.float32),
        acc2, y2, dinv, b2, batch.astype(jnp.int32)[None, :],
        global_features, Wg, bg, Wh1, bh1, Wh2, bh2)
    return q


# speedup vs baseline: 32.6910x; 1.0219x over previous
import functools

import jax
import jax.numpy as jnp
from jax import lax
from jax.experimental import pallas as pl
from jax.experimental.pallas import tpu as pltpu
from jax.experimental.pallas import tpu_sc as plsc

N = 10000
E = 320000
HID = 128
NACT = 64
NB = 64
NCORES = 2
NSUB = 16
NW = NCORES * NSUB
EPW = E // NW
CHUNK = 80
NCHUNK = EPW // CHUNK
NSTAGE = 5
STAGE = NCHUNK // NSTAGE
RPT = 632
NPAD = RPT * NSUB

_mesh = plsc.VectorSubcoreMesh(core_axis_name="c", subcore_axis_name="s")


@functools.partial(
    pl.kernel,
    out_type=jax.ShapeDtypeStruct((NW, N), jnp.float32),
    mesh=_mesh,
    scratch_types=[
        pltpu.VMEM((STAGE, CHUNK), jnp.int32),
        pltpu.VMEM((N,), jnp.float32),
    ],
    compiler_params=pltpu.CompilerParams(needs_layout_passes=False),
)
def _sc_degree(ei_hbm, out_hbm, idx_v, hist_v):
    cid = lax.axis_index("c")
    sid = lax.axis_index("s")
    wid = sid * NCORES + cid

    def zero_body(i, carry):
        hist_v[pl.ds(i * 16, 16)] = jnp.zeros((16,), jnp.float32)
        return carry

    lax.fori_loop(0, N // 16, zero_body, 0)

    ones = jnp.ones((16,), jnp.float32)

    def stage_body(st, carry):
        pltpu.sync_copy(ei_hbm.at[1, wid, st], idx_v)

        def body(c, cc):
            for k in range(CHUNK // 16):
                idx = idx_v[c, pl.ds(k * 16, 16)]
                plsc.addupdate_scatter(hist_v, [idx], ones)
            return cc

        lax.fori_loop(0, STAGE, body, 0)
        return carry

    lax.fori_loop(0, NSTAGE, stage_body, 0)
    pltpu.sync_copy(hist_v, out_hbm.at[wid])


@functools.partial(
    pl.kernel,
    out_type=jax.ShapeDtypeStruct((NCORES, NPAD, HID), jnp.float32),
    mesh=_mesh,
    scratch_types=[
        pltpu.VMEM((STAGE, CHUNK), jnp.int32),
        pltpu.VMEM((STAGE, CHUNK), jnp.int32),
        [pltpu.VMEM((CHUNK, HID), jnp.float32) for _ in range(3)],
        pltpu.VMEM_SHARED((NPAD, HID), jnp.float32),
        [pltpu.SemaphoreType.DMA for _ in range(3)],
        [pltpu.SemaphoreType.DMA for _ in range(3)],
    ],
)
def _sc_edge_agg(y_hbm, ei_hbm, zeros_hbm, out_hbm,
                 src_v, dst_v, rows, accum_sh, gsem, ssem):
    cid = lax.axis_index("c")
    sid = lax.axis_index("s")
    wid = sid * NCORES + cid

    r0 = sid * RPT
    pltpu.sync_copy(zeros_hbm.at[pl.ds(r0, RPT)], accum_sh.at[pl.ds(r0, RPT)])
    plsc.subcore_barrier()

    def gather(i, b):
        pltpu.async_copy(y_hbm.at[src_v.at[i]], rows[b], gsem[b])

    def gather_wait(i, b):
        pltpu.make_async_copy(y_hbm.at[src_v.at[i]], rows[b], gsem[b]).wait()

    def scatter(i, b):
        pltpu.async_copy(rows[b], accum_sh.at[dst_v.at[i]], ssem[b], add=True)

    def scatter_wait(i, b):
        pltpu.make_async_copy(rows[b], accum_sh.at[dst_v.at[i]],
                              ssem[b]).wait()

    def stage_body(s, carry):
        pltpu.sync_copy(ei_hbm.at[0, wid, s], src_v)
        pltpu.sync_copy(ei_hbm.at[1, wid, s], dst_v)

        gather(0, 0)
        gather(1, 1)
        gather_wait(0, 0)
        scatter(0, 0)
        gather(2, 2)

        def step(i, b):
            gather_wait(i, b)
            scatter(i, b)
            bp = (b + 2) % 3
            scatter_wait(i - 1, bp)
            gather(i + 2, bp)

        def body(j, c):
            for t in range(3):
                i = j * 3 + 1 + t
                step(i, (1 + t) % 3)
            return c

        lax.fori_loop(0, (STAGE - 4) // 3, body, 0)
        step(STAGE - 3, (STAGE - 3) % 3)
        for i in (STAGE - 2, STAGE - 1):
            b = i % 3
            gather_wait(i, b)
            scatter(i, b)
            scatter_wait(i - 1, (i - 1) % 3)
        scatter_wait(STAGE - 1, (STAGE - 1) % 3)
        return carry

    lax.fori_loop(0, NSTAGE, stage_body, 0)

    plsc.subcore_barrier()
    pltpu.sync_copy(accum_sh.at[pl.ds(r0, RPT)],
                    out_hbm.at[cid].at[pl.ds(r0, RPT)])


def _tc_pre_body(degp_ref, x_ref, wemb_ref, bemb_ref, w1_ref, dinv_ref, y_ref):
    deg = jnp.sum(degp_ref[...], axis=0) + 1.0
    dinv = lax.rsqrt(deg)
    dinv_ref[...] = dinv[:, None]
    h0 = jnp.maximum(
        jnp.dot(x_ref[...], wemb_ref[...],
                preferred_element_type=jnp.float32) + bemb_ref[...][None, :],
        0.0)
    xw = jnp.dot(h0, w1_ref[...], preferred_element_type=jnp.float32)
    y_ref[...] = xw * dinv[:, None]


def _tc_mid_body(acc_ref, yprev_ref, dinv_ref, bprev_ref, wnext_ref, y_ref):
    dinv = dinv_ref[...]
    acc = (acc_ref[0] + acc_ref[1])[:N]
    h = jnp.maximum(
        dinv * (acc + yprev_ref[...]) + bprev_ref[...][None, :], 0.0)
    y_ref[...] = jnp.dot(h, wnext_ref[...],
                         preferred_element_type=jnp.float32) * dinv


def _tc_final_body(acc_ref, yprev_ref, dinv_ref, b2_ref, batch_ref, gf_ref,
                   wg_ref, bg_ref, wh1_ref, bh1_ref, wh2_ref, bh2_ref, q_ref):
    dinv = dinv_ref[...]
    acc = (acc_ref[0] + acc_ref[1])[:N]
    h = jnp.maximum(
        dinv * (acc + yprev_ref[...]) + b2_ref[...][None, :],
        0.0)
    gid = lax.broadcasted_iota(jnp.int32, (NB, N), 0)
    oh_t = (gid == batch_ref[...]).astype(jnp.float32)
    seg = jnp.dot(oh_t, h, preferred_element_type=jnp.float32)
    cnt = jnp.sum(oh_t, axis=1)
    ge = seg / jnp.maximum(cnt, 1.0)[:, None]
    gl = jnp.maximum(
        jnp.dot(gf_ref[...], wg_ref[...],
                preferred_element_type=jnp.float32) + bg_ref[...][None, :],
        0.0)
    comb = jnp.concatenate([ge, gl], axis=1)
    hh = jnp.maximum(
        jnp.dot(comb, wh1_ref[...],
                preferred_element_type=jnp.float32) + bh1_ref[...][None, :],
        0.0)
    q_ref[...] = (jnp.dot(hh, wh2_ref[...], preferred_element_type=jnp.float32)
                  + bh2_ref[...][None, :])


def _tc_call(body, out_shape, *args):
    return pl.pallas_call(body, out_shape=out_shape)(*args)


def kernel(x, edge_index, batch, global_features, W_emb, b_emb, W1, b1,
           W2, b2, Wg, bg, Wh1, bh1, Wh2, bh2):
    ei = edge_index.astype(jnp.int32).reshape(2, NW, NSTAGE, STAGE, CHUNK)
    zeros = jnp.zeros((NPAD, HID), jnp.float32)

    deg_parts = _sc_degree(ei)

    dinv, y1 = _tc_call(
        _tc_pre_body,
        (jax.ShapeDtypeStruct((N, 1), jnp.float32),
         jax.ShapeDtypeStruct((N, HID), jnp.float32)),
        deg_parts, x, W_emb, b_emb, W1)

    acc1 = _sc_edge_agg(y1, ei, zeros)

    y2 = _tc_call(
        _tc_mid_body,
        jax.ShapeDtypeStruct((N, HID), jnp.float32),
        acc1, y1, dinv, b1, W2)

    acc2 = _sc_edge_agg(y2, ei, zeros)

    q = _tc_call(
        _tc_final_body,
        jax.ShapeDtypeStruct((NB, NACT), jnp.float32),
        acc2, y2, dinv, b2, batch.astype(jnp.int32)[None, :],
        global_features, Wg, bg, Wh1, bh1, Wh2, bh2)
    return q


# speedup vs baseline: 33.3727x; 1.0209x over previous
import functools

import jax
import jax.numpy as jnp
from jax import lax
from jax.experimental import pallas as pl
from jax.experimental.pallas import tpu as pltpu
from jax.experimental.pallas import tpu_sc as plsc

N = 10000
E = 320000
HID = 128
NACT = 64
NB = 64
NCORES = 2
NSUB = 16
NW = NCORES * NSUB
EPW = E // NW
CHUNK = 80
NCHUNK = EPW // CHUNK
NSTAGE = 5
STAGE = NCHUNK // NSTAGE
RPT = 632
NPAD = RPT * NSUB

_mesh = plsc.VectorSubcoreMesh(core_axis_name="c", subcore_axis_name="s")


@functools.partial(
    pl.kernel,
    out_type=jax.ShapeDtypeStruct((NW * NPAD,), jnp.float32),
    mesh=_mesh,
    scratch_types=[
        pltpu.VMEM((EPW,), jnp.int32),
        pltpu.VMEM((N,), jnp.float32),
    ],
    compiler_params=pltpu.CompilerParams(needs_layout_passes=False),
)
def _sc_degree(ei_hbm, out_hbm, idx_v, hist_v):
    cid = lax.axis_index("c")
    sid = lax.axis_index("s")
    wid = sid * NCORES + cid

    pltpu.sync_copy(ei_hbm.at[pl.ds(E + wid * EPW, EPW)], idx_v)

    def zero_body(i, carry):
        hist_v[pl.ds(i * 16, 16)] = jnp.zeros((16,), jnp.float32)
        return carry

    lax.fori_loop(0, N // 16, zero_body, 0)

    ones = jnp.ones((16,), jnp.float32)

    def body(i, carry):
        idx = idx_v[pl.ds(i * 16, 16)]
        plsc.addupdate_scatter(hist_v, [idx], ones)
        return carry

    lax.fori_loop(0, EPW // 16, body, 0)
    pltpu.sync_copy(hist_v, out_hbm.at[pl.ds(wid * NPAD, N)])


@functools.partial(
    pl.kernel,
    out_type=jax.ShapeDtypeStruct((NCORES, NPAD, HID), jnp.float32),
    mesh=_mesh,
    scratch_types=[
        pltpu.VMEM((STAGE * CHUNK,), jnp.int32),
        pltpu.VMEM((STAGE * CHUNK,), jnp.int32),
        [pltpu.VMEM((CHUNK, HID), jnp.float32) for _ in range(3)],
        pltpu.VMEM_SHARED((NPAD, HID), jnp.float32),
        [pltpu.SemaphoreType.DMA for _ in range(3)],
        [pltpu.SemaphoreType.DMA for _ in range(3)],
    ],
)
def _sc_edge_agg(y_hbm, ei_hbm, out_hbm,
                 src_v, dst_v, rows, accum_sh, gsem, ssem):
    cid = lax.axis_index("c")
    sid = lax.axis_index("s")
    wid = sid * NCORES + cid

    r0 = sid * RPT

    @pl.when(sid < NSUB - 1)
    def _():
        pltpu.sync_copy(y_hbm.at[pl.ds(r0, RPT)], accum_sh.at[pl.ds(r0, RPT)])

    @pl.when(sid == NSUB - 1)
    def _():
        pltpu.sync_copy(y_hbm.at[pl.ds((NSUB - 1) * RPT, N - (NSUB - 1) * RPT)],
                        accum_sh.at[pl.ds((NSUB - 1) * RPT,
                                          N - (NSUB - 1) * RPT)])

    plsc.subcore_barrier()

    def gather(i, b):
        pltpu.async_copy(y_hbm.at[src_v.at[pl.ds(i * CHUNK, CHUNK)]],
                         rows[b], gsem[b])

    def gather_wait(i, b):
        pltpu.make_async_copy(y_hbm.at[src_v.at[pl.ds(i * CHUNK, CHUNK)]],
                              rows[b], gsem[b]).wait()

    def scatter(i, b):
        pltpu.async_copy(rows[b], accum_sh.at[dst_v.at[pl.ds(i * CHUNK,
                                                             CHUNK)]],
                         ssem[b], add=True)

    def scatter_wait(i, b):
        pltpu.make_async_copy(rows[b], accum_sh.at[dst_v.at[pl.ds(i * CHUNK,
                                                                  CHUNK)]],
                              ssem[b]).wait()

    def stage_body(s, carry):
        e0 = wid * EPW + s * (STAGE * CHUNK)
        pltpu.sync_copy(ei_hbm.at[pl.ds(e0, STAGE * CHUNK)], src_v)
        pltpu.sync_copy(ei_hbm.at[pl.ds(E + e0, STAGE * CHUNK)], dst_v)

        gather(0, 0)
        gather(1, 1)
        gather_wait(0, 0)
        scatter(0, 0)
        gather(2, 2)

        def step(i, b):
            gather_wait(i, b)
            scatter(i, b)
            bp = (b + 2) % 3
            scatter_wait(i - 1, bp)
            gather(i + 2, bp)

        def body(j, c):
            for t in range(3):
                i = j * 3 + 1 + t
                step(i, (1 + t) % 3)
            return c

        lax.fori_loop(0, (STAGE - 4) // 3, body, 0)
        step(STAGE - 3, (STAGE - 3) % 3)
        for i in (STAGE - 2, STAGE - 1):
            b = i % 3
            gather_wait(i, b)
            scatter(i, b)
            scatter_wait(i - 1, (i - 1) % 3)
        scatter_wait(STAGE - 1, (STAGE - 1) % 3)
        return carry

    lax.fori_loop(0, NSTAGE, stage_body, 0)

    plsc.subcore_barrier()
    pltpu.sync_copy(accum_sh.at[pl.ds(r0, RPT)],
                    out_hbm.at[cid].at[pl.ds(r0, RPT)])


def _tc_pre_body(degp_ref, x_ref, wemb_ref, bemb_ref, w1_ref, dinv_ref, y_ref):
    deg = jnp.sum(degp_ref[...], axis=0)[:N] + 1.0
    dinv = lax.rsqrt(deg)
    dinv_ref[...] = dinv[:, None]
    h0 = jnp.maximum(
        jnp.dot(x_ref[...], wemb_ref[...],
                preferred_element_type=jnp.float32) + bemb_ref[...][None, :],
        0.0)
    xw = jnp.dot(h0, w1_ref[...], preferred_element_type=jnp.float32)
    y_ref[...] = xw * dinv[:, None]


def _tc_mid_body(acc_ref, yprev_ref, dinv_ref, bprev_ref, wnext_ref, y_ref):
    dinv = dinv_ref[...]
    acc = (acc_ref[0] + acc_ref[1])[:N]
    h = jnp.maximum(
        dinv * (acc - yprev_ref[...]) + bprev_ref[...][None, :], 0.0)
    y_ref[...] = jnp.dot(h, wnext_ref[...],
                         preferred_element_type=jnp.float32) * dinv


def _tc_final_body(acc_ref, yprev_ref, dinv_ref, b2_ref, batch_ref, gf_ref,
                   wg_ref, bg_ref, wh1_ref, bh1_ref, wh2_ref, bh2_ref, q_ref):
    dinv = dinv_ref[...]
    acc = (acc_ref[0] + acc_ref[1])[:N]
    h = jnp.maximum(
        dinv * (acc - yprev_ref[...]) + b2_ref[...][None, :],
        0.0)
    gid = lax.broadcasted_iota(jnp.int32, (NB, N), 0)
    oh_t = (gid == batch_ref[...]).astype(jnp.float32)
    seg = jnp.dot(oh_t, h, preferred_element_type=jnp.float32)
    cnt = jnp.sum(oh_t, axis=1)
    ge = seg / jnp.maximum(cnt, 1.0)[:, None]
    gl = jnp.maximum(
        jnp.dot(gf_ref[...], wg_ref[...],
                preferred_element_type=jnp.float32) + bg_ref[...][None, :],
        0.0)
    comb = jnp.concatenate([ge, gl], axis=1)
    hh = jnp.maximum(
        jnp.dot(comb, wh1_ref[...],
                preferred_element_type=jnp.float32) + bh1_ref[...][None, :],
        0.0)
    q_ref[...] = (jnp.dot(hh, wh2_ref[...], preferred_element_type=jnp.float32)
                  + bh2_ref[...][None, :])


def _tc_call(body, out_shape, *args):
    return pl.pallas_call(body, out_shape=out_shape)(*args)


def kernel(x, edge_index, batch, global_features, W_emb, b_emb, W1, b1,
           W2, b2, Wg, bg, Wh1, bh1, Wh2, bh2):
    ei = edge_index.astype(jnp.int32).reshape(2 * E)

    deg_parts = _sc_degree(ei).reshape(NW, NPAD)

    dinv, y1 = _tc_call(
        _tc_pre_body,
        (jax.ShapeDtypeStruct((N, 1), jnp.float32),
         jax.ShapeDtypeStruct((N, HID), jnp.float32)),
        deg_parts, x, W_emb, b_emb, W1)

    acc1 = _sc_edge_agg(y1, ei)

    y2 = _tc_call(
        _tc_mid_body,
        jax.ShapeDtypeStruct((N, HID), jnp.float32),
        acc1, y1, dinv, b1, W2)

    acc2 = _sc_edge_agg(y2, ei)

    q = _tc_call(
        _tc_final_body,
        jax.ShapeDtypeStruct((NB, NACT), jnp.float32),
        acc2, y2, dinv, b2, batch.astype(jnp.int32)[None, :],
        global_features, Wg, bg, Wh1, bh1, Wh2, bh2)
    return q
